# baseline (speedup 1.0000x reference)
import functools

import jax
import jax.numpy as jnp
from jax import lax
from jax.experimental import pallas as pl
from jax.experimental.pallas import tpu as pltpu
from jax.experimental.pallas import tpu_sc as plsc

N = 10000
E = 320000
D = 128

NC = 2
NS = 16
NW = NC * NS
L = 16

EB = 128
NBLK_TOTAL = E // EB
BPT = NBLK_TOTAL // NW
NTAIL = NBLK_TOTAL - BPT * NW

ZB = 80
NCHUNK = N // ZB
KMAX = (NCHUNK + NS - 1) // NS

EB2 = 40
BPT2 = E // (EB2 * NW)

ZB3 = 40
NCHUNK3 = N // ZB3
KMAX3 = (NCHUNK3 + NS - 1) // NS

_mesh = plsc.VectorSubcoreMesh(
    core_axis_name="c", subcore_axis_name="s", num_cores=NC, num_subcores=NS
)
_sc_params = pltpu.CompilerParams(needs_layout_passes=False)


@functools.partial(
    pl.kernel,
    out_type=[jax.ShapeDtypeStruct((N,), jnp.float32) for _ in range(4)],
    mesh=_mesh,
    scratch_types=[
        pltpu.VMEM((BPT, EB), jnp.int32),
        pltpu.VMEM((BPT, EB), jnp.int32),
        pltpu.VMEM((1, EB), jnp.int32),
        pltpu.VMEM((1, EB), jnp.int32),
        pltpu.VMEM((EB,), jnp.float32),
        pltpu.VMEM((ZB,), jnp.float32),
        pltpu.VMEM_SHARED((N,), jnp.float32),
        pltpu.VMEM_SHARED((N,), jnp.float32),
        pltpu.SemaphoreType.DMA,
        pltpu.SemaphoreType.DMA,
        pltpu.SemaphoreType.DMA,
        pltpu.SemaphoreType.DMA,
    ],
    compiler_params=_sc_params,
)
def _k1_degrees(rowm_hbm, colm_hbm, rowt_hbm, colt_hbm,
                degi0_hbm, degj0_hbm, degi1_hbm, degj1_hbm,
                ridx_v, cidx_v, tri_v, tci_v, ones_v, zero_v,
                degi_sp, degj_sp, sr0, sr1, sc0, sc1):
    cid = lax.axis_index("c")
    sid = lax.axis_index("s")
    wid = sid * NC + cid
    s_r = [sr0, sr1]
    s_c = [sc0, sc1]

    pltpu.sync_copy(rowm_hbm.at[wid], ridx_v)
    pltpu.sync_copy(colm_hbm.at[wid], cidx_v)

    @pl.when(wid < NTAIL)
    def _():
        pltpu.sync_copy(rowt_hbm.at[wid], tri_v)
        pltpu.sync_copy(colt_hbm.at[wid], tci_v)

    for i in range(EB // L):
        ones_v[pl.ds(i * L, L)] = jnp.ones((L,), jnp.float32)
    for i in range(ZB // L):
        zero_v[pl.ds(i * L, L)] = jnp.zeros((L,), jnp.float32)

    for k in range(KMAX):
        c = sid + NS * k

        @pl.when(c < NCHUNK)
        def _():
            pltpu.sync_copy(zero_v, degi_sp.at[pl.ds(c * ZB, ZB)])
            pltpu.sync_copy(zero_v, degj_sp.at[pl.ds(c * ZB, ZB)])

    plsc.subcore_barrier()

    def blk_body(k, carry):
        for p in range(2):
            b = 2 * k + p

            @pl.when(b >= 2)
            def _():
                pltpu.make_async_copy(
                    ones_v, degi_sp.at[ridx_v.at[0]], s_r[p]).wait()
                pltpu.make_async_copy(
                    ones_v, degj_sp.at[cidx_v.at[0]], s_c[p]).wait()

            pltpu.async_copy(ones_v, degi_sp.at[ridx_v.at[b]], s_r[p],
                             add=True)
            pltpu.async_copy(ones_v, degj_sp.at[cidx_v.at[b]], s_c[p],
                             add=True)
        return carry

    lax.fori_loop(0, BPT // 2, blk_body, None)
    for p in range(2):
        pltpu.make_async_copy(ones_v, degi_sp.at[ridx_v.at[0]], s_r[p]).wait()
        pltpu.make_async_copy(ones_v, degj_sp.at[cidx_v.at[0]], s_c[p]).wait()

    @pl.when(wid < NTAIL)
    def _():
        pltpu.sync_copy(ones_v, degi_sp.at[tri_v.at[0]], add=True)
        pltpu.sync_copy(ones_v, degj_sp.at[tci_v.at[0]], add=True)

    plsc.subcore_barrier()

    for cc, (di_hbm, dj_hbm) in enumerate(
        [(degi0_hbm, degj0_hbm), (degi1_hbm, degj1_hbm)]
    ):
        for k in range(KMAX):
            c = sid + NS * k

            @pl.when(jnp.logical_and(cid == cc, c < NCHUNK))
            def _():
                pltpu.sync_copy(degi_sp.at[pl.ds(c * ZB, ZB)], zero_v)
                pltpu.sync_copy(zero_v, di_hbm.at[pl.ds(c * ZB, ZB)])
                pltpu.sync_copy(degj_sp.at[pl.ds(c * ZB, ZB)], zero_v)
                pltpu.sync_copy(zero_v, dj_hbm.at[pl.ds(c * ZB, ZB)])


def _k2_body(x_ref, wt_ref, b_ref, deg_ref, xlin_ref, disj_ref):
    xlin_ref[...] = (
        jnp.dot(x_ref[...], wt_ref[...], preferred_element_type=jnp.float32)
        + b_ref[...]
    )
    degj = 1.0 + deg_ref[0, 1] + deg_ref[1, 1]
    disj_ref[...] = lax.rsqrt(degj)


def _k2_linear_disj(x, wt, b2d, deg4):
    nb = 400
    grid = N // nb
    return pl.pallas_call(
        _k2_body,
        grid=(grid,),
        in_specs=[
            pl.BlockSpec((nb, D), lambda i: (i, 0)),
            pl.BlockSpec((D, D), lambda i: (0, 0)),
            pl.BlockSpec((1, D), lambda i: (0, 0)),
            pl.BlockSpec((NC, 2, nb, 1), lambda i: (0, 0, i, 0)),
        ],
        out_specs=[
            pl.BlockSpec((nb, D), lambda i: (i, 0)),
            pl.BlockSpec((nb, 1), lambda i: (i, 0)),
        ],
        out_shape=[
            jax.ShapeDtypeStruct((N, D), jnp.float32),
            jax.ShapeDtypeStruct((N, 1), jnp.float32),
        ],
    )(x, wt, b2d, deg4)


@functools.partial(
    pl.kernel,
    out_type=jax.ShapeDtypeStruct((E,), jnp.float32),
    mesh=_mesh,
    scratch_types=[
        pltpu.VMEM((E // NW,), jnp.int32),
        pltpu.VMEM((N,), jnp.float32),
        pltpu.VMEM((E // NW,), jnp.float32),
    ],
    compiler_params=_sc_params,
)
def _k2c_dje(col_hbm, disj_hbm, dje_hbm, cidx_v, disj_v, dje_v):
    cid = lax.axis_index("c")
    sid = lax.axis_index("s")
    wid = sid * NC + cid
    ept = E // NW

    pltpu.sync_copy(disj_hbm, disj_v)
    pltpu.sync_copy(col_hbm.at[pl.ds(wid * ept, ept)], cidx_v)

    def grp(g, carry):
        col16 = cidx_v[pl.ds(g * L, L)]
        dje_v[pl.ds(g * L, L)] = plsc.load_gather(disj_v, [col16])
        return carry

    lax.fori_loop(0, ept // L, grp, None)
    pltpu.sync_copy(dje_v, dje_hbm.at[pl.ds(wid * ept, ept)])


@functools.partial(
    pl.kernel,
    out_type=jax.ShapeDtypeStruct((NC, N, D), jnp.float32),
    mesh=_mesh,
    scratch_types=[
        pltpu.VMEM((1, EB2), jnp.int32),
        pltpu.VMEM((1, EB2), jnp.int32),
        pltpu.VMEM((1, EB2), jnp.int32),
        pltpu.VMEM((1, EB2), jnp.int32),
        pltpu.VMEM((EB2,), jnp.float32),
        pltpu.VMEM((EB2,), jnp.float32),
        pltpu.VMEM((EB2, D), jnp.float32),
        pltpu.VMEM((EB2, D), jnp.float32),
        pltpu.VMEM((EB2, D), jnp.float32),
        pltpu.VMEM((EB2, D), jnp.float32),
        pltpu.VMEM((EB2, D), jnp.float32),
        pltpu.VMEM((EB2, D), jnp.float32),
        pltpu.VMEM_SHARED((N, D), jnp.float32),
    ] + [pltpu.SemaphoreType.DMA] * 12,
    compiler_params=_sc_params,
)
def _k3_edges(xlin_hbm, col4_hbm, row4_hbm, dje_hbm, ea_hbm, out_hbm,
              ri0, ri1, ci0, ci1, dj0, dj1,
              xr0, xr1, ea0, ea1, ms0, ms1, acc_sp,
              sri0, sri1, sci0, sci1, sdj0, sdj1,
              sg0, sg1, se0, se1, ss0, ss1):
    cid = lax.axis_index("c")
    sid = lax.axis_index("s")
    wid = sid * NC + cid
    ri = [ri0, ri1]
    ci = [ci0, ci1]
    dj = [dj0, dj1]
    xr = [xr0, xr1]
    ea = [ea0, ea1]
    ms = [ms0, ms1]
    s_ri = [sri0, sri1]
    s_ci = [sci0, sci1]
    s_dj = [sdj0, sdj1]
    s_g = [sg0, sg1]
    s_e = [se0, se1]
    s_s = [ss0, ss1]
    NB = BPT2

    def zrow(i, carry):
        for r in range(D // L):
            xr0[i, pl.ds(r * L, L)] = jnp.zeros((L,), jnp.float32)
        return carry

    lax.fori_loop(0, EB2, zrow, None)
    for k in range(KMAX3):
        c = sid + NS * k

        @pl.when(c < NCHUNK3)
        def _():
            pltpu.sync_copy(xr0.at[pl.ds(0, ZB3)],
                            acc_sp.at[pl.ds(c * ZB3, ZB3)])

    plsc.subcore_barrier()

    def compute_block(djref, xrref, msref, earef):
        for off in (0, L, EB2 - L):
            dj16 = djref[pl.ds(off, L)]
            for e in range(L):
                dj_b = jnp.take_along_axis(
                    dj16, jnp.full((L,), e, jnp.int32), axis=0
                )
                ei = off + e
                for r in range(D // L):
                    sl = pl.ds(r * L, L)
                    msref[ei, sl] = dj_b * (xrref[ei, sl] + earef[ei, sl])

    def issue_ri(blk, q):
        pltpu.async_copy(row4_hbm.at[wid, blk], ri[q], s_ri[q])

    def issue_ci(blk, q):
        pltpu.async_copy(col4_hbm.at[wid, blk], ci[q], s_ci[q])

    def issue_dje(blk, q):
        base = (wid * BPT2 + blk) * EB2
        pltpu.async_copy(dje_hbm.at[pl.ds(base, EB2)], dj[q], s_dj[q])

    def issue_ea(blk, q):
        base = (wid * BPT2 + blk) * EB2
        pltpu.async_copy(ea_hbm.at[pl.ds(base, EB2)], ea[q], s_e[q])

    def issue_gather(q, r):
        pltpu.async_copy(xlin_hbm.at[ci[r].at[0]], xr[q], s_g[q])

    def wait_ri(q):
        pltpu.make_async_copy(row4_hbm.at[wid, 0], ri[q], s_ri[q]).wait()

    def wait_ci(q):
        pltpu.make_async_copy(col4_hbm.at[wid, 0], ci[q], s_ci[q]).wait()

    def wait_dje(q):
        pltpu.make_async_copy(dje_hbm.at[pl.ds(0, EB2)], dj[q], s_dj[q]).wait()

    def wait_ea(q):
        pltpu.make_async_copy(ea_hbm.at[pl.ds(0, EB2)], ea[q], s_e[q]).wait()

    def wait_g(q):
        pltpu.make_async_copy(xlin_hbm.at[ci0.at[0]], xr[q], s_g[q]).wait()

    def wait_s(q):
        pltpu.make_async_copy(ms[q], acc_sp.at[ri0.at[0]], s_s[q]).wait()

    for q in range(2):
        issue_ri(q, q)
        issue_ci(q, q)
        issue_dje(q, q)
        issue_ea(q, q)
    wait_ci(0)
    issue_gather(0, 0)

    def blk_body(k, carry):
        for p in range(2):
            b = 2 * k + p
            q = 1 - p

            @pl.when(b + 1 <= NB - 1)
            def _():
                wait_ci(q)
                issue_gather(q, q)

            wait_g(p)
            wait_ea(p)
            wait_dje(p)
            wait_ri(p)
            compute_block(dj[p], xr[p], ms[p], ea[p])

            @pl.when(b >= 1)
            def _():
                wait_s(q)

            @pl.when(b + 1 <= NB - 1)
            def _():
                issue_ri(b + 1, q)

            pltpu.async_copy(ms[p], acc_sp.at[ri[p].at[0]], s_s[p], add=True)

            @pl.when(b + 2 <= NB - 1)
            def _():
                issue_ea(b + 2, p)
                issue_dje(b + 2, p)
                issue_ci(b + 2, p)
        return carry

    lax.fori_loop(0, NB // 2, blk_body, None)

    wait_s(1)

    plsc.subcore_barrier()

    for k in range(KMAX3):
        c = sid + NS * k

        @pl.when(c < NCHUNK3)
        def _():
            pltpu.sync_copy(acc_sp.at[pl.ds(c * ZB3, ZB3)],
                            xr0.at[pl.ds(0, ZB3)])
            pltpu.sync_copy(xr0.at[pl.ds(0, ZB3)],
                            out_hbm.at[cid, pl.ds(c * ZB3, ZB3)])


def _k4_body(acc_ref, xlin_ref, deg_ref, root_ref, o_ref):
    degi = 1.0 + deg_ref[0, 0] + deg_ref[1, 0]
    degj = 1.0 + deg_ref[0, 1] + deg_ref[1, 1]
    di = lax.rsqrt(degi)
    dj = lax.rsqrt(degj)
    s = (acc_ref[0] + acc_ref[1]) * di
    xl = xlin_ref[...]
    o_ref[...] = jnp.maximum(s, 0.0) + jnp.maximum(xl + root_ref[...], 0.0) * (
        di * dj
    )


def _k4_combine(acc, xlin, deg4, root2d):
    nb = 400
    grid = N // nb
    return pl.pallas_call(
        _k4_body,
        grid=(grid,),
        in_specs=[
            pl.BlockSpec((NC, nb, D), lambda i: (0, i, 0)),
            pl.BlockSpec((nb, D), lambda i: (i, 0)),
            pl.BlockSpec((NC, 2, nb, 1), lambda i: (0, 0, i, 0)),
            pl.BlockSpec((1, D), lambda i: (0, 0)),
        ],
        out_specs=pl.BlockSpec((nb, D), lambda i: (i, 0)),
        out_shape=jax.ShapeDtypeStruct((N, D), jnp.float32),
    )(acc, xlin, deg4, root2d)


def kernel(x, edge_index, edge_attr, root_emb, W, b):
    row = edge_index[0].astype(jnp.int32)
    col = edge_index[1].astype(jnp.int32)
    nmain = NW * BPT * EB
    rowm = row[:nmain].reshape(NW, BPT, EB)
    colm = col[:nmain].reshape(NW, BPT, EB)
    rowt = row[nmain:].reshape(NTAIL, 1, EB)
    colt = col[nmain:].reshape(NTAIL, 1, EB)
    di0, dj0, di1, dj1 = _k1_degrees(rowm, colm, rowt, colt)
    deg4 = jnp.stack([jnp.stack([di0, dj0]), jnp.stack([di1, dj1])])
    deg4 = deg4.reshape(NC, 2, N, 1)
    xlin, disj = _k2_linear_disj(x, W.T, b.reshape(1, D), deg4)
    dje = _k2c_dje(col, disj.reshape(N))

    row4 = row.reshape(NW, BPT2, 1, EB2)
    col4 = col.reshape(NW, BPT2, 1, EB2)
    acc = _k3_edges(xlin, col4, row4, dje, edge_attr)
    return _k4_combine(acc, xlin, deg4, root_emb.reshape(1, D))


# speedup vs baseline: 1.0546x; 1.0546x over previous
import functools

import jax
import jax.numpy as jnp
from jax import lax
from jax.experimental import pallas as pl
from jax.experimental.pallas import tpu as pltpu
from jax.experimental.pallas import tpu_sc as plsc

N = 10000
E = 320000
D = 128

NC = 2
NS = 16
NW = NC * NS
L = 16

EB = 128
NBLK_TOTAL = E // EB
BPT = NBLK_TOTAL // NW
NTAIL = NBLK_TOTAL - BPT * NW

ZB = 80
NCHUNK = N // ZB
KMAX = (NCHUNK + NS - 1) // NS

EB2 = 40
BPT2 = E // (EB2 * NW)

ZB3 = 40
NCHUNK3 = N // ZB3
KMAX3 = (NCHUNK3 + NS - 1) // NS

_mesh = plsc.VectorSubcoreMesh(
    core_axis_name="c", subcore_axis_name="s", num_cores=NC, num_subcores=NS
)
_sc_params = pltpu.CompilerParams(needs_layout_passes=False)


@functools.partial(
    pl.kernel,
    out_type=[jax.ShapeDtypeStruct((N,), jnp.float32) for _ in range(4)],
    mesh=_mesh,
    scratch_types=[
        pltpu.VMEM((BPT, EB), jnp.int32),
        pltpu.VMEM((BPT, EB), jnp.int32),
        pltpu.VMEM((1, EB), jnp.int32),
        pltpu.VMEM((1, EB), jnp.int32),
        pltpu.VMEM((EB,), jnp.float32),
        pltpu.VMEM((ZB,), jnp.float32),
        pltpu.VMEM_SHARED((N,), jnp.float32),
        pltpu.VMEM_SHARED((N,), jnp.float32),
        pltpu.SemaphoreType.DMA,
        pltpu.SemaphoreType.DMA,
        pltpu.SemaphoreType.DMA,
        pltpu.SemaphoreType.DMA,
    ],
    compiler_params=_sc_params,
)
def _k1_degrees(rowm_hbm, colm_hbm, rowt_hbm, colt_hbm,
                degi0_hbm, degj0_hbm, degi1_hbm, degj1_hbm,
                ridx_v, cidx_v, tri_v, tci_v, ones_v, zero_v,
                degi_sp, degj_sp, sr0, sr1, sc0, sc1):
    cid = lax.axis_index("c")
    sid = lax.axis_index("s")
    wid = sid * NC + cid
    s_r = [sr0, sr1]
    s_c = [sc0, sc1]

    pltpu.sync_copy(rowm_hbm.at[wid], ridx_v)
    pltpu.sync_copy(colm_hbm.at[wid], cidx_v)

    @pl.when(wid < NTAIL)
    def _():
        pltpu.sync_copy(rowt_hbm.at[wid], tri_v)
        pltpu.sync_copy(colt_hbm.at[wid], tci_v)

    for i in range(EB // L):
        ones_v[pl.ds(i * L, L)] = jnp.ones((L,), jnp.float32)
    for i in range(ZB // L):
        zero_v[pl.ds(i * L, L)] = jnp.zeros((L,), jnp.float32)

    for k in range(KMAX):
        c = sid + NS * k

        @pl.when(c < NCHUNK)
        def _():
            pltpu.sync_copy(zero_v, degi_sp.at[pl.ds(c * ZB, ZB)])
            pltpu.sync_copy(zero_v, degj_sp.at[pl.ds(c * ZB, ZB)])

    plsc.subcore_barrier()

    def blk_body(k, carry):
        for p in range(2):
            b = 2 * k + p

            @pl.when(b >= 2)
            def _():
                pltpu.make_async_copy(
                    ones_v, degi_sp.at[ridx_v.at[0]], s_r[p]).wait()
                pltpu.make_async_copy(
                    ones_v, degj_sp.at[cidx_v.at[0]], s_c[p]).wait()

            pltpu.async_copy(ones_v, degi_sp.at[ridx_v.at[b]], s_r[p],
                             add=True)
            pltpu.async_copy(ones_v, degj_sp.at[cidx_v.at[b]], s_c[p],
                             add=True)
        return carry

    lax.fori_loop(0, BPT // 2, blk_body, None)
    for p in range(2):
        pltpu.make_async_copy(ones_v, degi_sp.at[ridx_v.at[0]], s_r[p]).wait()
        pltpu.make_async_copy(ones_v, degj_sp.at[cidx_v.at[0]], s_c[p]).wait()

    @pl.when(wid < NTAIL)
    def _():
        pltpu.sync_copy(ones_v, degi_sp.at[tri_v.at[0]], add=True)
        pltpu.sync_copy(ones_v, degj_sp.at[tci_v.at[0]], add=True)

    plsc.subcore_barrier()

    for cc, (di_hbm, dj_hbm) in enumerate(
        [(degi0_hbm, degj0_hbm), (degi1_hbm, degj1_hbm)]
    ):
        for k in range(KMAX):
            c = sid + NS * k

            @pl.when(jnp.logical_and(cid == cc, c < NCHUNK))
            def _():
                pltpu.sync_copy(degi_sp.at[pl.ds(c * ZB, ZB)], zero_v)
                pltpu.sync_copy(zero_v, di_hbm.at[pl.ds(c * ZB, ZB)])
                pltpu.sync_copy(degj_sp.at[pl.ds(c * ZB, ZB)], zero_v)
                pltpu.sync_copy(zero_v, dj_hbm.at[pl.ds(c * ZB, ZB)])


def _k2_body(x_ref, wt_ref, b_ref, dj0_ref, dj1_ref, xlin_ref, disj_ref):
    xlin_ref[...] = (
        jnp.dot(x_ref[...], wt_ref[...], preferred_element_type=jnp.float32)
        + b_ref[...]
    )
    degj = 1.0 + dj0_ref[...] + dj1_ref[...]
    disj_ref[...] = lax.rsqrt(degj)


def _k2_linear_disj(x, wt, b2d, dj0, dj1):
    nb = 400
    grid = N // nb
    return pl.pallas_call(
        _k2_body,
        grid=(grid,),
        in_specs=[
            pl.BlockSpec((nb, D), lambda i: (i, 0)),
            pl.BlockSpec((D, D), lambda i: (0, 0)),
            pl.BlockSpec((1, D), lambda i: (0, 0)),
            pl.BlockSpec((nb, 1), lambda i: (i, 0)),
            pl.BlockSpec((nb, 1), lambda i: (i, 0)),
        ],
        out_specs=[
            pl.BlockSpec((nb, D), lambda i: (i, 0)),
            pl.BlockSpec((nb, 1), lambda i: (i, 0)),
        ],
        out_shape=[
            jax.ShapeDtypeStruct((N, D), jnp.float32),
            jax.ShapeDtypeStruct((N, 1), jnp.float32),
        ],
    )(x, wt, b2d, dj0, dj1)


@functools.partial(
    pl.kernel,
    out_type=jax.ShapeDtypeStruct((E,), jnp.float32),
    mesh=_mesh,
    scratch_types=[
        pltpu.VMEM((E // NW,), jnp.int32),
        pltpu.VMEM((N,), jnp.float32),
        pltpu.VMEM((E // NW,), jnp.float32),
    ],
    compiler_params=_sc_params,
)
def _k2c_dje(col_hbm, disj_hbm, dje_hbm, cidx_v, disj_v, dje_v):
    cid = lax.axis_index("c")
    sid = lax.axis_index("s")
    wid = sid * NC + cid
    ept = E // NW

    pltpu.sync_copy(disj_hbm, disj_v)
    pltpu.sync_copy(col_hbm.at[pl.ds(wid * ept, ept)], cidx_v)

    def grp(g, carry):
        col16 = cidx_v[pl.ds(g * L, L)]
        dje_v[pl.ds(g * L, L)] = plsc.load_gather(disj_v, [col16])
        return carry

    lax.fori_loop(0, ept // L, grp, None)
    pltpu.sync_copy(dje_v, dje_hbm.at[pl.ds(wid * ept, ept)])


@functools.partial(
    pl.kernel,
    out_type=jax.ShapeDtypeStruct((NC, N, D), jnp.float32),
    mesh=_mesh,
    scratch_types=[
        pltpu.VMEM((1, EB2), jnp.int32),
        pltpu.VMEM((1, EB2), jnp.int32),
        pltpu.VMEM((1, EB2), jnp.int32),
        pltpu.VMEM((1, EB2), jnp.int32),
        pltpu.VMEM((EB2 + 8,), jnp.float32),
        pltpu.VMEM((EB2 + 8,), jnp.float32),
        pltpu.VMEM((EB2, D), jnp.float32),
        pltpu.VMEM((EB2, D), jnp.float32),
        pltpu.VMEM((EB2, D), jnp.float32),
        pltpu.VMEM((EB2, D), jnp.float32),
        pltpu.VMEM((EB2, D), jnp.float32),
        pltpu.VMEM((EB2, D), jnp.float32),
        pltpu.VMEM_SHARED((N, D), jnp.float32),
    ] + [pltpu.SemaphoreType.DMA] * 12,
    compiler_params=_sc_params,
)
def _k3_edges(xlin_hbm, col4_hbm, row4_hbm, dje_hbm, ea_hbm, out_hbm,
              ri0, ri1, ci0, ci1, dj0, dj1,
              xr0, xr1, ea0, ea1, ms0, ms1, acc_sp,
              sri0, sri1, sci0, sci1, sdj0, sdj1,
              sg0, sg1, se0, se1, ss0, ss1):
    cid = lax.axis_index("c")
    sid = lax.axis_index("s")
    wid = sid * NC + cid
    ri = [ri0, ri1]
    ci = [ci0, ci1]
    dj = [dj0, dj1]
    xr = [xr0, xr1]
    ea = [ea0, ea1]
    ms = [ms0, ms1]
    s_ri = [sri0, sri1]
    s_ci = [sci0, sci1]
    s_dj = [sdj0, sdj1]
    s_g = [sg0, sg1]
    s_e = [se0, se1]
    s_s = [ss0, ss1]
    NB = BPT2

    def zrow(i, carry):
        for r in range(D // L):
            xr0[i, pl.ds(r * L, L)] = jnp.zeros((L,), jnp.float32)
        return carry

    lax.fori_loop(0, EB2, zrow, None)
    for k in range(KMAX3):
        c = sid + NS * k

        @pl.when(c < NCHUNK3)
        def _():
            pltpu.sync_copy(xr0.at[pl.ds(0, ZB3)],
                            acc_sp.at[pl.ds(c * ZB3, ZB3)])

    plsc.subcore_barrier()

    def compute_block(djref, xrref, msref, earef):
        for off in range(0, EB2, L):
            dj16 = djref[pl.ds(off, L)]
            for e in range(min(L, EB2 - off)):
                dj_b = jnp.take_along_axis(
                    dj16, jnp.full((L,), e, jnp.int32), axis=0
                )
                ei = off + e
                for r in range(D // L):
                    sl = pl.ds(r * L, L)
                    msref[ei, sl] = dj_b * (xrref[ei, sl] + earef[ei, sl])

    def issue_ri(blk, q):
        pltpu.async_copy(row4_hbm.at[wid, blk], ri[q], s_ri[q])

    def issue_ci(blk, q):
        pltpu.async_copy(col4_hbm.at[wid, blk], ci[q], s_ci[q])

    def issue_dje(blk, q):
        base = (wid * BPT2 + blk) * EB2
        pltpu.async_copy(dje_hbm.at[pl.ds(base, EB2)], dj[q].at[pl.ds(0, EB2)],
                         s_dj[q])

    def issue_ea(blk, q):
        base = (wid * BPT2 + blk) * EB2
        pltpu.async_copy(ea_hbm.at[pl.ds(base, EB2)], ea[q], s_e[q])

    def issue_gather(q, r):
        pltpu.async_copy(xlin_hbm.at[ci[r].at[0]], xr[q], s_g[q])

    def wait_ri(q):
        pltpu.make_async_copy(row4_hbm.at[wid, 0], ri[q], s_ri[q]).wait()

    def wait_ci(q):
        pltpu.make_async_copy(col4_hbm.at[wid, 0], ci[q], s_ci[q]).wait()

    def wait_dje(q):
        pltpu.make_async_copy(dje_hbm.at[pl.ds(0, EB2)],
                              dj[q].at[pl.ds(0, EB2)], s_dj[q]).wait()

    def wait_ea(q):
        pltpu.make_async_copy(ea_hbm.at[pl.ds(0, EB2)], ea[q], s_e[q]).wait()

    def wait_g(q):
        pltpu.make_async_copy(xlin_hbm.at[ci0.at[0]], xr[q], s_g[q]).wait()

    def wait_s(q):
        pltpu.make_async_copy(ms[q], acc_sp.at[ri0.at[0]], s_s[q]).wait()

    for q in range(2):
        issue_ri(q, q)
        issue_ci(q, q)
        issue_dje(q, q)
        issue_ea(q, q)
    wait_ci(0)
    issue_gather(0, 0)

    def blk_body(k, carry):
        for p in range(2):
            b = 2 * k + p
            q = 1 - p

            @pl.when(b + 1 <= NB - 1)
            def _():
                wait_ci(q)
                issue_gather(q, q)

            wait_g(p)
            wait_ea(p)
            wait_dje(p)
            wait_ri(p)
            compute_block(dj[p], xr[p], ms[p], ea[p])

            @pl.when(b >= 1)
            def _():
                wait_s(q)

            @pl.when(b + 1 <= NB - 1)
            def _():
                issue_ri(b + 1, q)

            pltpu.async_copy(ms[p], acc_sp.at[ri[p].at[0]], s_s[p], add=True)

            @pl.when(b + 2 <= NB - 1)
            def _():
                issue_ea(b + 2, p)
                issue_dje(b + 2, p)
                issue_ci(b + 2, p)
        return carry

    lax.fori_loop(0, NB // 2, blk_body, None)

    wait_s(1)

    plsc.subcore_barrier()

    for k in range(KMAX3):
        c = sid + NS * k

        @pl.when(c < NCHUNK3)
        def _():
            pltpu.sync_copy(acc_sp.at[pl.ds(c * ZB3, ZB3)],
                            xr0.at[pl.ds(0, ZB3)])
            pltpu.sync_copy(xr0.at[pl.ds(0, ZB3)],
                            out_hbm.at[cid, pl.ds(c * ZB3, ZB3)])


def _k4_body(acc_ref, xlin_ref, di0_ref, di1_ref, dj0_ref, dj1_ref,
             root_ref, o_ref):
    degi = 1.0 + di0_ref[...] + di1_ref[...]
    degj = 1.0 + dj0_ref[...] + dj1_ref[...]
    di = lax.rsqrt(degi)
    dj = lax.rsqrt(degj)
    s = (acc_ref[0] + acc_ref[1]) * di
    xl = xlin_ref[...]
    o_ref[...] = jnp.maximum(s, 0.0) + jnp.maximum(xl + root_ref[...], 0.0) * (
        di * dj
    )


def _k4_combine(acc, xlin, degs, root2d):
    nb = 400
    grid = N // nb
    return pl.pallas_call(
        _k4_body,
        grid=(grid,),
        in_specs=[
            pl.BlockSpec((NC, nb, D), lambda i: (0, i, 0)),
            pl.BlockSpec((nb, D), lambda i: (i, 0)),
            pl.BlockSpec((nb, 1), lambda i: (i, 0)),
            pl.BlockSpec((nb, 1), lambda i: (i, 0)),
            pl.BlockSpec((nb, 1), lambda i: (i, 0)),
            pl.BlockSpec((nb, 1), lambda i: (i, 0)),
            pl.BlockSpec((1, D), lambda i: (0, 0)),
        ],
        out_specs=pl.BlockSpec((nb, D), lambda i: (i, 0)),
        out_shape=jax.ShapeDtypeStruct((N, D), jnp.float32),
    )(acc, xlin, *degs, root2d)


def kernel(x, edge_index, edge_attr, root_emb, W, b):
    row = edge_index[0].astype(jnp.int32)
    col = edge_index[1].astype(jnp.int32)
    nmain = NW * BPT * EB
    rowm = row[:nmain].reshape(NW, BPT, EB)
    colm = col[:nmain].reshape(NW, BPT, EB)
    rowt = row[nmain:].reshape(NTAIL, 1, EB)
    colt = col[nmain:].reshape(NTAIL, 1, EB)
    di0, dj0, di1, dj1 = _k1_degrees(rowm, colm, rowt, colt)
    di0, dj0, di1, dj1 = (v.reshape(N, 1) for v in (di0, dj0, di1, dj1))
    xlin, disj = _k2_linear_disj(x, W.T, b.reshape(1, D), dj0, dj1)
    dje = _k2c_dje(col, disj.reshape(N))

    row4 = row.reshape(NW, BPT2, 1, EB2)
    col4 = col.reshape(NW, BPT2, 1, EB2)
    acc = _k3_edges(xlin, col4, row4, dje, edge_attr)
    return _k4_combine(acc, xlin, (di0, di1, dj0, dj1),
                       root_emb.reshape(1, D))


# speedup vs baseline: 1.2094x; 1.1467x over previous
import functools

import jax
import jax.numpy as jnp
from jax import lax
from jax.experimental import pallas as pl
from jax.experimental.pallas import tpu as pltpu
from jax.experimental.pallas import tpu_sc as plsc

N = 10000
E = 320000
D = 128

NC = 2
NS = 16
NW = NC * NS
L = 16

EB = 128
NBLK_TOTAL = E // EB
BPT = NBLK_TOTAL // NW
NTAIL = NBLK_TOTAL - BPT * NW

ZB = 80
NCHUNK = N // ZB
KMAX = (NCHUNK + NS - 1) // NS

EB2 = 40
BPT2 = E // (EB2 * NW)

ZB3 = 40
NCHUNK3 = N // ZB3
KMAX3 = (NCHUNK3 + NS - 1) // NS

_mesh = plsc.VectorSubcoreMesh(
    core_axis_name="c", subcore_axis_name="s", num_cores=NC, num_subcores=NS
)
_sc_params = pltpu.CompilerParams(needs_layout_passes=False)


@functools.partial(
    pl.kernel,
    out_type=[jax.ShapeDtypeStruct((N,), jnp.float32) for _ in range(4)],
    mesh=_mesh,
    scratch_types=[
        pltpu.VMEM((BPT, EB), jnp.int32),
        pltpu.VMEM((BPT, EB), jnp.int32),
        pltpu.VMEM((1, EB), jnp.int32),
        pltpu.VMEM((1, EB), jnp.int32),
        pltpu.VMEM((EB,), jnp.float32),
        pltpu.VMEM((ZB,), jnp.float32),
        pltpu.VMEM_SHARED((N,), jnp.float32),
        pltpu.VMEM_SHARED((N,), jnp.float32),
        pltpu.SemaphoreType.DMA,
        pltpu.SemaphoreType.DMA,
        pltpu.SemaphoreType.DMA,
        pltpu.SemaphoreType.DMA,
    ],
    compiler_params=_sc_params,
)
def _k1_degrees(rowm_hbm, colm_hbm, rowt_hbm, colt_hbm,
                degi0_hbm, degj0_hbm, degi1_hbm, degj1_hbm,
                ridx_v, cidx_v, tri_v, tci_v, ones_v, zero_v,
                degi_sp, degj_sp, sr0, sr1, sc0, sc1):
    cid = lax.axis_index("c")
    sid = lax.axis_index("s")
    wid = sid * NC + cid
    s_r = [sr0, sr1]
    s_c = [sc0, sc1]

    pltpu.sync_copy(rowm_hbm.at[wid], ridx_v)
    pltpu.sync_copy(colm_hbm.at[wid], cidx_v)

    @pl.when(wid < NTAIL)
    def _():
        pltpu.sync_copy(rowt_hbm.at[wid], tri_v)
        pltpu.sync_copy(colt_hbm.at[wid], tci_v)

    for i in range(EB // L):
        ones_v[pl.ds(i * L, L)] = jnp.ones((L,), jnp.float32)
    for i in range(ZB // L):
        zero_v[pl.ds(i * L, L)] = jnp.zeros((L,), jnp.float32)

    for k in range(KMAX):
        c = sid + NS * k

        @pl.when(c < NCHUNK)
        def _():
            pltpu.sync_copy(zero_v, degi_sp.at[pl.ds(c * ZB, ZB)])
            pltpu.sync_copy(zero_v, degj_sp.at[pl.ds(c * ZB, ZB)])

    plsc.subcore_barrier()

    def blk_body(k, carry):
        for p in range(2):
            b = 2 * k + p

            @pl.when(b >= 2)
            def _():
                pltpu.make_async_copy(
                    ones_v, degi_sp.at[ridx_v.at[0]], s_r[p]).wait()
                pltpu.make_async_copy(
                    ones_v, degj_sp.at[cidx_v.at[0]], s_c[p]).wait()

            pltpu.async_copy(ones_v, degi_sp.at[ridx_v.at[b]], s_r[p],
                             add=True)
            pltpu.async_copy(ones_v, degj_sp.at[cidx_v.at[b]], s_c[p],
                             add=True)
        return carry

    lax.fori_loop(0, BPT // 2, blk_body, None)
    for p in range(2):
        pltpu.make_async_copy(ones_v, degi_sp.at[ridx_v.at[0]], s_r[p]).wait()
        pltpu.make_async_copy(ones_v, degj_sp.at[cidx_v.at[0]], s_c[p]).wait()

    @pl.when(wid < NTAIL)
    def _():
        pltpu.sync_copy(ones_v, degi_sp.at[tri_v.at[0]], add=True)
        pltpu.sync_copy(ones_v, degj_sp.at[tci_v.at[0]], add=True)

    plsc.subcore_barrier()

    for cc, (di_hbm, dj_hbm) in enumerate(
        [(degi0_hbm, degj0_hbm), (degi1_hbm, degj1_hbm)]
    ):
        for k in range(KMAX):
            c = sid + NS * k

            @pl.when(jnp.logical_and(cid == cc, c < NCHUNK))
            def _():
                pltpu.sync_copy(degi_sp.at[pl.ds(c * ZB, ZB)], zero_v)
                pltpu.sync_copy(zero_v, di_hbm.at[pl.ds(c * ZB, ZB)])
                pltpu.sync_copy(degj_sp.at[pl.ds(c * ZB, ZB)], zero_v)
                pltpu.sync_copy(zero_v, dj_hbm.at[pl.ds(c * ZB, ZB)])


def _k2_body(x_ref, wt_ref, b_ref, dj0_ref, dj1_ref, xlin_ref, disj_ref):
    xlin_ref[...] = (
        jnp.dot(x_ref[...], wt_ref[...], preferred_element_type=jnp.float32)
        + b_ref[...]
    )
    degj = 1.0 + dj0_ref[...] + dj1_ref[...]
    disj_ref[...] = lax.rsqrt(degj)


def _k2_linear_disj(x, wt, b2d, dj0, dj1):
    nb = 400
    grid = N // nb
    return pl.pallas_call(
        _k2_body,
        grid=(grid,),
        in_specs=[
            pl.BlockSpec((nb, D), lambda i: (i, 0)),
            pl.BlockSpec((D, D), lambda i: (0, 0)),
            pl.BlockSpec((1, D), lambda i: (0, 0)),
            pl.BlockSpec((nb, 1), lambda i: (i, 0)),
            pl.BlockSpec((nb, 1), lambda i: (i, 0)),
        ],
        out_specs=[
            pl.BlockSpec((nb, D), lambda i: (i, 0)),
            pl.BlockSpec((nb, 1), lambda i: (i, 0)),
        ],
        out_shape=[
            jax.ShapeDtypeStruct((N, D), jnp.float32),
            jax.ShapeDtypeStruct((N, 1), jnp.float32),
        ],
    )(x, wt, b2d, dj0, dj1)


@functools.partial(
    pl.kernel,
    out_type=jax.ShapeDtypeStruct((E,), jnp.float32),
    mesh=_mesh,
    scratch_types=[
        pltpu.VMEM((E // NW,), jnp.int32),
        pltpu.VMEM((N,), jnp.float32),
        pltpu.VMEM((E // NW,), jnp.float32),
    ],
    compiler_params=_sc_params,
)
def _k2c_dje(col_hbm, disj_hbm, dje_hbm, cidx_v, disj_v, dje_v):
    cid = lax.axis_index("c")
    sid = lax.axis_index("s")
    wid = sid * NC + cid
    ept = E // NW

    pltpu.sync_copy(disj_hbm, disj_v)
    pltpu.sync_copy(col_hbm.at[pl.ds(wid * ept, ept)], cidx_v)

    def grp(g, carry):
        col16 = cidx_v[pl.ds(g * L, L)]
        dje_v[pl.ds(g * L, L)] = plsc.load_gather(disj_v, [col16])
        return carry

    lax.fori_loop(0, ept // L, grp, None)
    pltpu.sync_copy(dje_v, dje_hbm.at[pl.ds(wid * ept, ept)])


@functools.partial(
    pl.kernel,
    out_type=jax.ShapeDtypeStruct((NC, N, D), jnp.float32),
    mesh=_mesh,
    scratch_types=[
        pltpu.VMEM((1, EB2), jnp.int32),
        pltpu.VMEM((1, EB2), jnp.int32),
        pltpu.VMEM((1, EB2), jnp.int32),
        pltpu.VMEM((1, EB2), jnp.int32),
        pltpu.VMEM((N,), jnp.float32),
        pltpu.VMEM((EB2, D), jnp.float32),
        pltpu.VMEM((EB2, D), jnp.float32),
        pltpu.VMEM((EB2, D), jnp.float32),
        pltpu.VMEM((EB2, D), jnp.float32),
        pltpu.VMEM((EB2, D), jnp.float32),
        pltpu.VMEM((EB2, D), jnp.float32),
        pltpu.VMEM_SHARED((N, D), jnp.float32),
    ] + [pltpu.SemaphoreType.DMA] * 10,
    compiler_params=_sc_params,
)
def _k3_edges(xlin_hbm, col4_hbm, row4_hbm, disj_hbm, ea_hbm, out_hbm,
              ri0, ri1, ci0, ci1, disj_v,
              xr0, xr1, ea0, ea1, ms0, ms1, acc_sp,
              sri0, sri1, sci0, sci1,
              sg0, sg1, se0, se1, ss0, ss1):
    cid = lax.axis_index("c")
    sid = lax.axis_index("s")
    wid = sid * NC + cid
    ri = [ri0, ri1]
    ci = [ci0, ci1]
    xr = [xr0, xr1]
    ea = [ea0, ea1]
    ms = [ms0, ms1]
    s_ri = [sri0, sri1]
    s_ci = [sci0, sci1]
    s_g = [sg0, sg1]
    s_e = [se0, se1]
    s_s = [ss0, ss1]
    NB = BPT2

    pltpu.sync_copy(disj_hbm, disj_v)

    def zrow(i, carry):
        for r in range(D // L):
            xr0[i, pl.ds(r * L, L)] = jnp.zeros((L,), jnp.float32)
        return carry

    lax.fori_loop(0, EB2, zrow, None)
    for k in range(KMAX3):
        c = sid + NS * k

        @pl.when(c < NCHUNK3)
        def _():
            pltpu.sync_copy(xr0.at[pl.ds(0, ZB3)],
                            acc_sp.at[pl.ds(c * ZB3, ZB3)])

    plsc.subcore_barrier()

    def compute_block(ciref, xrref, msref, earef):
        for off, e0 in ((0, 0), (L, 0), (EB2 - L, L - (EB2 - 2 * L))):
            col16 = ciref[0, pl.ds(off, L)]
            dj16 = plsc.load_gather(disj_v, [col16])
            for e in range(e0, L):
                dj_b = jnp.take_along_axis(
                    dj16, jnp.full((L,), e, jnp.int32), axis=0
                )
                ei = off + e
                for r in range(D // L):
                    sl = pl.ds(r * L, L)
                    msref[ei, sl] = dj_b * (xrref[ei, sl] + earef[ei, sl])

    def issue_ri(blk, q):
        pltpu.async_copy(row4_hbm.at[wid, blk], ri[q], s_ri[q])

    def issue_ci(blk, q):
        pltpu.async_copy(col4_hbm.at[wid, blk], ci[q], s_ci[q])

    def issue_ea(blk, q):
        base = (wid * BPT2 + blk) * EB2
        pltpu.async_copy(ea_hbm.at[pl.ds(base, EB2)], ea[q], s_e[q])

    def issue_gather(q, r):
        pltpu.async_copy(xlin_hbm.at[ci[r].at[0]], xr[q], s_g[q])

    def wait_ri(q):
        pltpu.make_async_copy(row4_hbm.at[wid, 0], ri[q], s_ri[q]).wait()

    def wait_ci(q):
        pltpu.make_async_copy(col4_hbm.at[wid, 0], ci[q], s_ci[q]).wait()

    def wait_ea(q):
        pltpu.make_async_copy(ea_hbm.at[pl.ds(0, EB2)], ea[q], s_e[q]).wait()

    def wait_g(q):
        pltpu.make_async_copy(xlin_hbm.at[ci0.at[0]], xr[q], s_g[q]).wait()

    def wait_s(q):
        pltpu.make_async_copy(ms[q], acc_sp.at[ri0.at[0]], s_s[q]).wait()

    for q in range(2):
        issue_ri(q, q)
        issue_ci(q, q)
        issue_ea(q, q)
    wait_ci(0)
    issue_gather(0, 0)

    def blk_body(k, carry):
        for p in range(2):
            b = 2 * k + p
            q = 1 - p

            @pl.when(b + 1 <= NB - 1)
            def _():
                wait_ci(q)
                issue_gather(q, q)

            wait_g(p)
            wait_ea(p)
            wait_ri(p)
            compute_block(ci[p], xr[p], ms[p], ea[p])

            @pl.when(b >= 1)
            def _():
                wait_s(q)

            @pl.when(b + 1 <= NB - 1)
            def _():
                issue_ri(b + 1, q)

            pltpu.async_copy(ms[p], acc_sp.at[ri[p].at[0]], s_s[p], add=True)

            @pl.when(b + 2 <= NB - 1)
            def _():
                issue_ea(b + 2, p)
                issue_ci(b + 2, p)
        return carry

    lax.fori_loop(0, NB // 2, blk_body, None)

    wait_s(1)

    plsc.subcore_barrier()

    for k in range(KMAX3):
        c = sid + NS * k

        @pl.when(c < NCHUNK3)
        def _():
            pltpu.sync_copy(acc_sp.at[pl.ds(c * ZB3, ZB3)],
                            xr0.at[pl.ds(0, ZB3)])
            pltpu.sync_copy(xr0.at[pl.ds(0, ZB3)],
                            out_hbm.at[cid, pl.ds(c * ZB3, ZB3)])


def _k4_body(acc_ref, xlin_ref, di0_ref, di1_ref, dj0_ref, dj1_ref,
             root_ref, o_ref):
    degi = 1.0 + di0_ref[...] + di1_ref[...]
    degj = 1.0 + dj0_ref[...] + dj1_ref[...]
    di = lax.rsqrt(degi)
    dj = lax.rsqrt(degj)
    s = (acc_ref[0] + acc_ref[1]) * di
    xl = xlin_ref[...]
    o_ref[...] = jnp.maximum(s, 0.0) + jnp.maximum(xl + root_ref[...], 0.0) * (
        di * dj
    )


def _k4_combine(acc, xlin, degs, root2d):
    nb = 400
    grid = N // nb
    return pl.pallas_call(
        _k4_body,
        grid=(grid,),
        in_specs=[
            pl.BlockSpec((NC, nb, D), lambda i: (0, i, 0)),
            pl.BlockSpec((nb, D), lambda i: (i, 0)),
            pl.BlockSpec((nb, 1), lambda i: (i, 0)),
            pl.BlockSpec((nb, 1), lambda i: (i, 0)),
            pl.BlockSpec((nb, 1), lambda i: (i, 0)),
            pl.BlockSpec((nb, 1), lambda i: (i, 0)),
            pl.BlockSpec((1, D), lambda i: (0, 0)),
        ],
        out_specs=pl.BlockSpec((nb, D), lambda i: (i, 0)),
        out_shape=jax.ShapeDtypeStruct((N, D), jnp.float32),
    )(acc, xlin, *degs, root2d)


def kernel(x, edge_index, edge_attr, root_emb, W, b):
    row = edge_index[0].astype(jnp.int32)
    col = edge_index[1].astype(jnp.int32)
    nmain = NW * BPT * EB
    rowm = row[:nmain].reshape(NW, BPT, EB)
    colm = col[:nmain].reshape(NW, BPT, EB)
    rowt = row[nmain:].reshape(NTAIL, 1, EB)
    colt = col[nmain:].reshape(NTAIL, 1, EB)
    di0, dj0, di1, dj1 = _k1_degrees(rowm, colm, rowt, colt)
    di0, dj0, di1, dj1 = (v.reshape(N, 1) for v in (di0, dj0, di1, dj1))
    xlin, disj = _k2_linear_disj(x, W.T, b.reshape(1, D), dj0, dj1)

    row4 = row.reshape(NW, BPT2, 1, EB2)
    col4 = col.reshape(NW, BPT2, 1, EB2)
    acc = _k3_edges(xlin, col4, row4, disj.reshape(N),
                    edge_attr)
    return _k4_combine(acc, xlin, (di0, di1, dj0, dj1),
                       root_emb.reshape(1, D))


# speedup vs baseline: 1.2113x; 1.0016x over previous
import functools

import jax
import jax.numpy as jnp
from jax import lax
from jax.experimental import pallas as pl
from jax.experimental.pallas import tpu as pltpu
from jax.experimental.pallas import tpu_sc as plsc

N = 10000
E = 320000
D = 128

NC = 2
NS = 16
NW = NC * NS
L = 16

EB = 128
NBLK_TOTAL = E // EB
BPT = NBLK_TOTAL // NW
NTAIL = NBLK_TOTAL - BPT * NW

ZB = 80
NCHUNK = N // ZB
KMAX = (NCHUNK + NS - 1) // NS

EB2 = 40
BPT2 = E // (EB2 * NW)

ZB3 = 40
NCHUNK3 = N // ZB3
KMAX3 = (NCHUNK3 + NS - 1) // NS

_mesh = plsc.VectorSubcoreMesh(
    core_axis_name="c", subcore_axis_name="s", num_cores=NC, num_subcores=NS
)
_sc_params = pltpu.CompilerParams(needs_layout_passes=False)


@functools.partial(
    pl.kernel,
    out_type=[jax.ShapeDtypeStruct((N,), jnp.float32) for _ in range(4)],
    mesh=_mesh,
    scratch_types=[
        pltpu.VMEM((BPT, EB), jnp.int32),
        pltpu.VMEM((BPT, EB), jnp.int32),
        pltpu.VMEM((1, EB), jnp.int32),
        pltpu.VMEM((1, EB), jnp.int32),
        pltpu.VMEM((EB,), jnp.float32),
        pltpu.VMEM((ZB,), jnp.float32),
        pltpu.VMEM_SHARED((N,), jnp.float32),
        pltpu.VMEM_SHARED((N,), jnp.float32),
        pltpu.SemaphoreType.DMA,
        pltpu.SemaphoreType.DMA,
        pltpu.SemaphoreType.DMA,
        pltpu.SemaphoreType.DMA,
    ],
    compiler_params=_sc_params,
)
def _k1_degrees(rowm_hbm, colm_hbm, rowt_hbm, colt_hbm,
                degi0_hbm, degj0_hbm, degi1_hbm, degj1_hbm,
                ridx_v, cidx_v, tri_v, tci_v, ones_v, zero_v,
                degi_sp, degj_sp, sr0, sr1, sc0, sc1):
    cid = lax.axis_index("c")
    sid = lax.axis_index("s")
    wid = sid * NC + cid
    s_r = [sr0, sr1]
    s_c = [sc0, sc1]

    pltpu.sync_copy(rowm_hbm.at[wid], ridx_v)
    pltpu.sync_copy(colm_hbm.at[wid], cidx_v)

    @pl.when(wid < NTAIL)
    def _():
        pltpu.sync_copy(rowt_hbm.at[wid], tri_v)
        pltpu.sync_copy(colt_hbm.at[wid], tci_v)

    for i in range(EB // L):
        ones_v[pl.ds(i * L, L)] = jnp.ones((L,), jnp.float32)
    for i in range(ZB // L):
        zero_v[pl.ds(i * L, L)] = jnp.zeros((L,), jnp.float32)

    for k in range(KMAX):
        c = sid + NS * k

        @pl.when(c < NCHUNK)
        def _():
            pltpu.sync_copy(zero_v, degi_sp.at[pl.ds(c * ZB, ZB)])
            pltpu.sync_copy(zero_v, degj_sp.at[pl.ds(c * ZB, ZB)])

    plsc.subcore_barrier()

    def blk_body(k, carry):
        for p in range(2):
            b = 2 * k + p

            @pl.when(b >= 2)
            def _():
                pltpu.make_async_copy(
                    ones_v, degi_sp.at[ridx_v.at[0]], s_r[p]).wait()
                pltpu.make_async_copy(
                    ones_v, degj_sp.at[cidx_v.at[0]], s_c[p]).wait()

            pltpu.async_copy(ones_v, degi_sp.at[ridx_v.at[b]], s_r[p],
                             add=True)
            pltpu.async_copy(ones_v, degj_sp.at[cidx_v.at[b]], s_c[p],
                             add=True)
        return carry

    lax.fori_loop(0, BPT // 2, blk_body, None)
    for p in range(2):
        pltpu.make_async_copy(ones_v, degi_sp.at[ridx_v.at[0]], s_r[p]).wait()
        pltpu.make_async_copy(ones_v, degj_sp.at[cidx_v.at[0]], s_c[p]).wait()

    @pl.when(wid < NTAIL)
    def _():
        pltpu.sync_copy(ones_v, degi_sp.at[tri_v.at[0]], add=True)
        pltpu.sync_copy(ones_v, degj_sp.at[tci_v.at[0]], add=True)

    plsc.subcore_barrier()

    for cc, (di_hbm, dj_hbm) in enumerate(
        [(degi0_hbm, degj0_hbm), (degi1_hbm, degj1_hbm)]
    ):
        for k in range(KMAX):
            c = sid + NS * k

            @pl.when(jnp.logical_and(cid == cc, c < NCHUNK))
            def _():
                pltpu.sync_copy(degi_sp.at[pl.ds(c * ZB, ZB)], zero_v)
                pltpu.sync_copy(zero_v, di_hbm.at[pl.ds(c * ZB, ZB)])
                pltpu.sync_copy(degj_sp.at[pl.ds(c * ZB, ZB)], zero_v)
                pltpu.sync_copy(zero_v, dj_hbm.at[pl.ds(c * ZB, ZB)])


def _k2_body(x_ref, w_ref, b_ref, dj0_ref, dj1_ref, xlin_ref, disj_ref):
    xw = lax.dot_general(
        x_ref[...], w_ref[...], (((1,), (1,)), ((), ())),
        preferred_element_type=jnp.float32,
    )
    xlin_ref[...] = xw + b_ref[...]
    degj = 1.0 + dj0_ref[...] + dj1_ref[...]
    disj_ref[...] = lax.rsqrt(degj)


def _k2_linear_disj(x, w, b2d, dj0, dj1):
    nb = 400
    grid = N // nb
    return pl.pallas_call(
        _k2_body,
        grid=(grid,),
        in_specs=[
            pl.BlockSpec((nb, D), lambda i: (i, 0)),
            pl.BlockSpec((D, D), lambda i: (0, 0)),
            pl.BlockSpec((1, D), lambda i: (0, 0)),
            pl.BlockSpec((nb, 1), lambda i: (i, 0)),
            pl.BlockSpec((nb, 1), lambda i: (i, 0)),
        ],
        out_specs=[
            pl.BlockSpec((nb, D), lambda i: (i, 0)),
            pl.BlockSpec((nb, 1), lambda i: (i, 0)),
        ],
        out_shape=[
            jax.ShapeDtypeStruct((N, D), jnp.float32),
            jax.ShapeDtypeStruct((N, 1), jnp.float32),
        ],
    )(x, w, b2d, dj0, dj1)


@functools.partial(
    pl.kernel,
    out_type=jax.ShapeDtypeStruct((E,), jnp.float32),
    mesh=_mesh,
    scratch_types=[
        pltpu.VMEM((E // NW,), jnp.int32),
        pltpu.VMEM((N,), jnp.float32),
        pltpu.VMEM((E // NW,), jnp.float32),
    ],
    compiler_params=_sc_params,
)
def _k2c_dje(col_hbm, disj_hbm, dje_hbm, cidx_v, disj_v, dje_v):
    cid = lax.axis_index("c")
    sid = lax.axis_index("s")
    wid = sid * NC + cid
    ept = E // NW

    pltpu.sync_copy(disj_hbm, disj_v)
    pltpu.sync_copy(col_hbm.at[pl.ds(wid * ept, ept)], cidx_v)

    def grp(g, carry):
        col16 = cidx_v[pl.ds(g * L, L)]
        dje_v[pl.ds(g * L, L)] = plsc.load_gather(disj_v, [col16])
        return carry

    lax.fori_loop(0, ept // L, grp, None)
    pltpu.sync_copy(dje_v, dje_hbm.at[pl.ds(wid * ept, ept)])


@functools.partial(
    pl.kernel,
    out_type=jax.ShapeDtypeStruct((NC, N, D), jnp.float32),
    mesh=_mesh,
    scratch_types=[
        pltpu.VMEM((1, EB2), jnp.int32),
        pltpu.VMEM((1, EB2), jnp.int32),
        pltpu.VMEM((1, EB2), jnp.int32),
        pltpu.VMEM((1, EB2), jnp.int32),
        pltpu.VMEM((N,), jnp.float32),
        pltpu.VMEM((EB2, D), jnp.float32),
        pltpu.VMEM((EB2, D), jnp.float32),
        pltpu.VMEM((EB2, D), jnp.float32),
        pltpu.VMEM((EB2, D), jnp.float32),
        pltpu.VMEM((EB2, D), jnp.float32),
        pltpu.VMEM((EB2, D), jnp.float32),
        pltpu.VMEM_SHARED((N, D), jnp.float32),
    ] + [pltpu.SemaphoreType.DMA] * 10,
    compiler_params=_sc_params,
)
def _k3_edges(xlin_hbm, col4_hbm, row4_hbm, disj_hbm, ea_hbm, out_hbm,
              ri0, ri1, ci0, ci1, disj_v,
              xr0, xr1, ea0, ea1, ms0, ms1, acc_sp,
              sri0, sri1, sci0, sci1,
              sg0, sg1, se0, se1, ss0, ss1):
    cid = lax.axis_index("c")
    sid = lax.axis_index("s")
    wid = sid * NC + cid
    ri = [ri0, ri1]
    ci = [ci0, ci1]
    xr = [xr0, xr1]
    ea = [ea0, ea1]
    ms = [ms0, ms1]
    s_ri = [sri0, sri1]
    s_ci = [sci0, sci1]
    s_g = [sg0, sg1]
    s_e = [se0, se1]
    s_s = [ss0, ss1]
    NB = BPT2

    pltpu.sync_copy(disj_hbm, disj_v)

    def zrow(i, carry):
        for r in range(D // L):
            xr0[i, pl.ds(r * L, L)] = jnp.zeros((L,), jnp.float32)
        return carry

    lax.fori_loop(0, EB2, zrow, None)
    for k in range(KMAX3):
        c = sid + NS * k

        @pl.when(c < NCHUNK3)
        def _():
            pltpu.sync_copy(xr0.at[pl.ds(0, ZB3)],
                            acc_sp.at[pl.ds(c * ZB3, ZB3)])

    plsc.subcore_barrier()

    def compute_block(ciref, xrref, msref, earef):
        for off, e0 in ((0, 0), (L, 0), (EB2 - L, L - (EB2 - 2 * L))):
            col16 = ciref[0, pl.ds(off, L)]
            dj16 = plsc.load_gather(disj_v, [col16])
            for e in range(e0, L):
                dj_b = jnp.take_along_axis(
                    dj16, jnp.full((L,), e, jnp.int32), axis=0
                )
                ei = off + e
                for r in range(D // L):
                    sl = pl.ds(r * L, L)
                    msref[ei, sl] = dj_b * (xrref[ei, sl] + earef[ei, sl])

    def issue_ri(blk, q):
        pltpu.async_copy(row4_hbm.at[wid, blk], ri[q], s_ri[q])

    def issue_ci(blk, q):
        pltpu.async_copy(col4_hbm.at[wid, blk], ci[q], s_ci[q])

    def issue_ea(blk, q):
        base = (wid * BPT2 + blk) * EB2
        pltpu.async_copy(ea_hbm.at[pl.ds(base, EB2)], ea[q], s_e[q])

    def issue_gather(q, r):
        pltpu.async_copy(xlin_hbm.at[ci[r].at[0]], xr[q], s_g[q])

    def wait_ri(q):
        pltpu.make_async_copy(row4_hbm.at[wid, 0], ri[q], s_ri[q]).wait()

    def wait_ci(q):
        pltpu.make_async_copy(col4_hbm.at[wid, 0], ci[q], s_ci[q]).wait()

    def wait_ea(q):
        pltpu.make_async_copy(ea_hbm.at[pl.ds(0, EB2)], ea[q], s_e[q]).wait()

    def wait_g(q):
        pltpu.make_async_copy(xlin_hbm.at[ci0.at[0]], xr[q], s_g[q]).wait()

    def wait_s(q):
        pltpu.make_async_copy(ms[q], acc_sp.at[ri0.at[0]], s_s[q]).wait()

    for q in range(2):
        issue_ri(q, q)
        issue_ci(q, q)
        issue_ea(q, q)
    wait_ci(0)
    issue_gather(0, 0)

    def blk_body(k, carry):
        for p in range(2):
            b = 2 * k + p
            q = 1 - p

            @pl.when(b + 1 <= NB - 1)
            def _():
                wait_ci(q)
                issue_gather(q, q)

            wait_g(p)
            wait_ea(p)
            wait_ri(p)
            compute_block(ci[p], xr[p], ms[p], ea[p])

            @pl.when(b >= 1)
            def _():
                wait_s(q)

            @pl.when(b + 1 <= NB - 1)
            def _():
                issue_ri(b + 1, q)

            pltpu.async_copy(ms[p], acc_sp.at[ri[p].at[0]], s_s[p], add=True)

            @pl.when(b + 2 <= NB - 1)
            def _():
                issue_ea(b + 2, p)
                issue_ci(b + 2, p)
        return carry

    lax.fori_loop(0, NB // 2, blk_body, None)

    wait_s(1)

    plsc.subcore_barrier()

    for k in range(KMAX3):
        c = sid + NS * k

        @pl.when(c < NCHUNK3)
        def _():
            pltpu.sync_copy(acc_sp.at[pl.ds(c * ZB3, ZB3)],
                            xr0.at[pl.ds(0, ZB3)])
            pltpu.sync_copy(xr0.at[pl.ds(0, ZB3)],
                            out_hbm.at[cid, pl.ds(c * ZB3, ZB3)])


def _k4_body(acc_ref, xlin_ref, di0_ref, di1_ref, dj0_ref, dj1_ref,
             root_ref, o_ref):
    degi = 1.0 + di0_ref[...] + di1_ref[...]
    degj = 1.0 + dj0_ref[...] + dj1_ref[...]
    di = lax.rsqrt(degi)
    dj = lax.rsqrt(degj)
    s = (acc_ref[0] + acc_ref[1]) * di
    xl = xlin_ref[...]
    o_ref[...] = jnp.maximum(s, 0.0) + jnp.maximum(xl + root_ref[...], 0.0) * (
        di * dj
    )


def _k4_combine(acc, xlin, degs, root2d):
    nb = 400
    grid = N // nb
    return pl.pallas_call(
        _k4_body,
        grid=(grid,),
        in_specs=[
            pl.BlockSpec((NC, nb, D), lambda i: (0, i, 0)),
            pl.BlockSpec((nb, D), lambda i: (i, 0)),
            pl.BlockSpec((nb, 1), lambda i: (i, 0)),
            pl.BlockSpec((nb, 1), lambda i: (i, 0)),
            pl.BlockSpec((nb, 1), lambda i: (i, 0)),
            pl.BlockSpec((nb, 1), lambda i: (i, 0)),
            pl.BlockSpec((1, D), lambda i: (0, 0)),
        ],
        out_specs=pl.BlockSpec((nb, D), lambda i: (i, 0)),
        out_shape=jax.ShapeDtypeStruct((N, D), jnp.float32),
    )(acc, xlin, *degs, root2d)


def kernel(x, edge_index, edge_attr, root_emb, W, b):
    row = edge_index[0].astype(jnp.int32)
    col = edge_index[1].astype(jnp.int32)
    nmain = NW * BPT * EB
    rowm = row[:nmain].reshape(NW, BPT, EB)
    colm = col[:nmain].reshape(NW, BPT, EB)
    rowt = row[nmain:].reshape(NTAIL, 1, EB)
    colt = col[nmain:].reshape(NTAIL, 1, EB)
    di0, dj0, di1, dj1 = _k1_degrees(rowm, colm, rowt, colt)
    di0, dj0, di1, dj1 = (v.reshape(N, 1) for v in (di0, dj0, di1, dj1))
    xlin, disj = _k2_linear_disj(x, W, b.reshape(1, D), dj0, dj1)

    row4 = row.reshape(NW, BPT2, 1, EB2)
    col4 = col.reshape(NW, BPT2, 1, EB2)
    acc = _k3_edges(xlin, col4, row4, disj.reshape(N),
                    edge_attr)
    return _k4_combine(acc, xlin, (di0, di1, dj0, dj1),
                       root_emb.reshape(1, D))
